# Initial kernel scaffold; baseline (speedup 1.0000x reference)
import jax
import jax.numpy as jnp
from jax.experimental import pallas as pl


def kernel(z_e_x, codebook):
    raise NotImplementedError("write your pallas kernel here")


# speedup vs baseline: 1.6216x; 1.6216x over previous
import functools

import jax
import jax.numpy as jnp
from jax import lax
from jax.experimental import pallas as pl
from jax.experimental.pallas import tpu as pltpu
from jax.experimental.pallas import tpu_sc as plsc

_ROWS = 1024

_NC = 2
_NS = 16
_NW = _NC * _NS


def _argmin_body(x_ref, insq_ref, cb_ref, cbsq_ref, idx_ref):
    x = x_ref[...]
    cb = cb_ref[...]
    mm = lax.dot_general(x, cb, (((1,), (1,)), ((), ())),
                         preferred_element_type=jnp.float32)
    d = (cbsq_ref[...] + insq_ref[...]) - 2.0 * mm
    k = d.shape[1]
    min_d = jnp.min(d, axis=1, keepdims=True)
    iota = lax.broadcasted_iota(jnp.int32, d.shape, 1)
    idx = jnp.min(jnp.where(d == min_d, iota, jnp.int32(k)), axis=1)
    idx_ref[...] = idx.reshape(-1, 1)


def _argmin_call(x, insq, codebook, cbsq):
    n, d_ = x.shape
    k = codebook.shape[0]
    return pl.pallas_call(
        _argmin_body,
        grid=(n // _ROWS,),
        in_specs=[
            pl.BlockSpec((_ROWS, d_), lambda i: (i, 0)),
            pl.BlockSpec((_ROWS, 1), lambda i: (i, 0)),
            pl.BlockSpec((k, d_), lambda i: (0, 0)),
            pl.BlockSpec((1, k), lambda i: (0, 0)),
        ],
        out_specs=pl.BlockSpec((_ROWS, 1), lambda i: (i, 0)),
        out_shape=jax.ShapeDtypeStruct((n, 1), jnp.int32),
    )(x, insq, codebook, cbsq)


@functools.lru_cache(maxsize=None)
def _make_gather(n, d_):
    b_per_w = n // _NW
    mesh = plsc.VectorSubcoreMesh(core_axis_name="c", subcore_axis_name="s")

    @functools.partial(
        pl.kernel, mesh=mesh,
        compiler_params=pltpu.CompilerParams(use_tc_tiling_on_sc=False),
        out_type=jax.ShapeDtypeStruct((n, d_), jnp.float32),
        scratch_types=[
            pltpu.VMEM((b_per_w,), jnp.int32),
            pltpu.VMEM((b_per_w, d_), jnp.float32),
            pltpu.SemaphoreType.DMA,
        ],
    )
    def gk(table_hbm, idx_hbm, out_hbm, idx_v, rows_v, sem):
        wid = lax.axis_index("s") * _NC + lax.axis_index("c")
        base = wid * b_per_w
        pltpu.sync_copy(idx_hbm.at[pl.ds(base, b_per_w)], idx_v)
        pltpu.async_copy(table_hbm.at[idx_v], rows_v, sem).wait()
        pltpu.sync_copy(rows_v, out_hbm.at[pl.ds(base, b_per_w)])

    return gk


def kernel(z_e_x, codebook):
    d_ = codebook.shape[1]
    x = z_e_x.reshape(-1, d_)
    insq = jnp.sum(x ** 2, axis=1, keepdims=True)
    cbsq = jnp.sum(codebook ** 2, axis=1)[None, :]
    idx2 = _argmin_call(x, insq, codebook, cbsq)
    indices = idx2.reshape(-1)
    z_q = _make_gather(x.shape[0], d_)(codebook, indices)
    z_q = z_q.reshape(z_e_x.shape)
    return (z_q, z_q, indices)
